# baseline (speedup 1.0000x reference)
import jax
import jax.numpy as jnp
from jax.experimental import pallas as pl


def _body(ea_ref, o_ref):
    o_ref[...] = jnp.sum(ea_ref[...], axis=0, keepdims=True)[:1, :1] * 0.0


def kernel(x, edge_attr, u, a, node_gid, edge_gid, We, Weg, be, Wn, Wng, bn,
           Wgn, Wge, bg, Wh, bh, Wo, bo):
    E, dE = edge_attr.shape
    out = pl.pallas_call(
        _body,
        grid=(10,),
        in_specs=[pl.BlockSpec((E // 10, dE), lambda i: (i, 0))],
        out_specs=pl.BlockSpec((1, 1), lambda i: (0, 0)),
        out_shape=jax.ShapeDtypeStruct((1, 1), jnp.float32),
    )(edge_attr)
    return out


# speedup vs baseline: 1.4876x; 1.4876x over previous
import functools

import jax
import jax.numpy as jnp
from jax.experimental import pallas as pl
from jax.experimental.pallas import tpu as pltpu


def _body(eaT_ref, egr_ref, x_ref, ngr_ref, ngc_ref, uT_ref, u_ref,
          We_ref, Weg_ref, beC_ref, WnT_ref, WngT_ref, bn_ref,
          WgnT_ref, WgeT_ref, bg_ref, a_ref, WhT_ref, bh_ref,
          WoT_ref, bo_ref, out_ref,
          esumT, ecnt, nsum, ncnt, *, G, B, EB, NB):
    i = pl.program_id(0)

    @pl.when(i == 0)
    def _():
        esumT[...] = jnp.zeros_like(esumT)
        ecnt[...] = jnp.zeros_like(ecnt)
        nsum[...] = jnp.zeros_like(nsum)
        ncnt[...] = jnp.zeros_like(ncnt)

    f32 = jnp.float32
    bf16 = jnp.bfloat16

    ugbT = (jnp.dot(Weg_ref[...], uT_ref[...], preferred_element_type=f32)
            + beC_ref[...]).astype(bf16)
    unb = (jnp.dot(u_ref[...], WngT_ref[...], preferred_element_type=f32)
           + bn_ref[...]).astype(bf16)

    ebT = eaT_ref[...]
    egr = egr_ref[0]
    ohr = (jax.lax.broadcasted_iota(jnp.int32, (B, EB), 0) == egr)
    ohr_b = ohr.astype(bf16)
    inT = jnp.concatenate([ebT, ohr_b], axis=0)
    Wfull = jnp.concatenate([We_ref[...], ugbT], axis=1)
    preT = jnp.dot(Wfull, inT, preferred_element_type=f32)
    e1T = jnp.maximum(preT, 0.0).astype(bf16)
    esumT[...] += jax.lax.dot_general(
        e1T, ohr_b, (((1,), (1,)), ((), ())), preferred_element_type=f32)
    ecnt[...] += jnp.sum(ohr.astype(f32), axis=1, keepdims=True)

    xb = x_ref[...]
    ngr = ngr_ref[0]
    nohr = (jax.lax.broadcasted_iota(jnp.int32, (B, NB), 0) == ngr)
    nohc = ngc_ref[...] == jax.lax.broadcasted_iota(
        jnp.int32, (NB, B), 1)
    xcat = jnp.concatenate([xb, nohc.astype(bf16)], axis=1)
    Wncat = jnp.concatenate([WnT_ref[...], unb], axis=0)
    npre = jnp.dot(xcat, Wncat, preferred_element_type=f32)
    n1 = jnp.maximum(npre, 0.0).astype(bf16)
    nsum[...] += jnp.dot(nohr.astype(bf16), n1, preferred_element_type=f32)
    ncnt[...] += jnp.sum(nohr.astype(f32), axis=1, keepdims=True)

    @pl.when(i == G - 1)
    def _():
        navg = nsum[...] / jnp.maximum(ncnt[...], 1.0)
        eavg = esumT[...].T / jnp.maximum(ecnt[...], 1.0)
        sv = (jnp.dot(navg, WgnT_ref[...], preferred_element_type=f32)
              + jnp.dot(eavg, WgeT_ref[...], preferred_element_type=f32)
              + bg_ref[...])
        sa = jnp.concatenate([sv, a_ref[...]], axis=1)
        h = jnp.maximum(jnp.dot(sa, WhT_ref[...], preferred_element_type=f32)
                        + bh_ref[...], 0.0)
        out_ref[...] = jnp.maximum(
            jnp.dot(h, WoT_ref[...], preferred_element_type=f32) + bo_ref[...], 0.0)


def kernel(x, edge_attr, u, a, node_gid, edge_gid, We, Weg, be, Wn, Wng, bn,
           Wgn, Wge, bg, Wh, bh, Wo, bo, *, interpret=False):
    N, dN = x.shape
    E, dE = edge_attr.shape
    B, dG = u.shape
    nA = a.shape[1]
    H = Wh.shape[0]
    G = 5
    assert E % G == 0 and N % G == 0
    EB = E // G
    NB = N // G

    f32 = jnp.float32
    bf16 = jnp.bfloat16
    eaT = edge_attr.T.astype(bf16)
    egid_row = edge_gid.reshape(G, 1, EB)
    ngid_row = node_gid.reshape(G, 1, NB)

    body = functools.partial(_body, G=G, B=B, EB=EB, NB=NB)
    const = lambda i: (0, 0)
    out = pl.pallas_call(
        body,
        grid=(G,),
        in_specs=[
            pl.BlockSpec((dE, EB), lambda i: (0, i)),
            pl.BlockSpec((1, 1, EB), lambda i: (i, 0, 0)),
            pl.BlockSpec((NB, dN), lambda i: (i, 0)),
            pl.BlockSpec((1, 1, NB), lambda i: (i, 0, 0)),
            pl.BlockSpec((NB, 1), lambda i: (i, 0)),
            pl.BlockSpec((dG, B), const),
            pl.BlockSpec((B, dG), const),
            pl.BlockSpec((128, dE), const),
            pl.BlockSpec((128, dG), const),
            pl.BlockSpec((128, 1), const),
            pl.BlockSpec((dN, 128), const),
            pl.BlockSpec((dG, 128), const),
            pl.BlockSpec((1, 128), const),
            pl.BlockSpec((128, 1), const),
            pl.BlockSpec((128, 1), const),
            pl.BlockSpec((1, 1), const),
            pl.BlockSpec((B, nA), const),
            pl.BlockSpec((1 + nA, H), const),
            pl.BlockSpec((1, H), const),
            pl.BlockSpec((H, 1), const),
            pl.BlockSpec((1, 1), const),
        ],
        out_specs=pl.BlockSpec((B, 1), const),
        out_shape=jax.ShapeDtypeStruct((B, 1), f32),
        scratch_shapes=[
            pltpu.VMEM((128, B), f32),
            pltpu.VMEM((B, 1), f32),
            pltpu.VMEM((B, 128), f32),
            pltpu.VMEM((B, 1), f32),
        ],
        compiler_params=pltpu.CompilerParams(
            dimension_semantics=("arbitrary",)),
        interpret=interpret,
    )(eaT, egid_row, x.astype(bf16), ngid_row, node_gid.reshape(N, 1), u.T, u,
      We.astype(bf16), Weg, be.reshape(128, 1),
      Wn.T.astype(bf16), Wng.T, bn.reshape(1, 128),
      Wgn.T, Wge.T, bg.reshape(1, 1), a, Wh.T, bh.reshape(1, H),
      Wo.T, bo.reshape(1, 1))
    return out


# speedup vs baseline: 1.4892x; 1.0011x over previous
import functools

import jax
import jax.numpy as jnp
from jax.experimental import pallas as pl
from jax.experimental.pallas import tpu as pltpu


def _body(eaT_ref, egr_ref, x_ref, ngr_ref, ngc_ref, uT_ref, u_ref,
          We_ref, Weg_ref, beC_ref, WnT_ref, WngT_ref, bn_ref,
          WgnT_ref, WgeT_ref, bg_ref, a_ref, WhT_ref, bh_ref,
          WoT_ref, bo_ref, out_ref,
          esumT, ecnt, nsum, ncnt, *, G, B, EB, NB):
    i = pl.program_id(0)

    @pl.when(i == 0)
    def _():
        esumT[...] = jnp.zeros_like(esumT)
        ecnt[...] = jnp.zeros_like(ecnt)
        nsum[...] = jnp.zeros_like(nsum)
        ncnt[...] = jnp.zeros_like(ncnt)

    f32 = jnp.float32
    bf16 = jnp.bfloat16

    ugbT = (jnp.dot(Weg_ref[...], uT_ref[...], preferred_element_type=f32)
            + beC_ref[...]).astype(bf16)
    unb = (jnp.dot(u_ref[...], WngT_ref[...], preferred_element_type=f32)
           + bn_ref[...]).astype(bf16)

    ebT = eaT_ref[...]
    egr = egr_ref[0]
    ohr = (jax.lax.broadcasted_iota(jnp.int32, (B, EB), 0) == egr)
    ohr_b = ohr.astype(bf16)
    inT = jnp.concatenate([ebT, ohr_b], axis=0)
    Wfull = jnp.concatenate([We_ref[...], ugbT], axis=1)
    preT = jnp.dot(Wfull, inT, preferred_element_type=f32)
    e1T = jnp.maximum(preT, 0.0).astype(bf16)
    esumT[...] += jax.lax.dot_general(
        ohr_b, e1T, (((1,), (1,)), ((), ())), preferred_element_type=f32)
    ecnt[...] += jnp.sum(ohr.astype(f32), axis=1, keepdims=True)

    xb = x_ref[...]
    ngr = ngr_ref[0]
    nohr = (jax.lax.broadcasted_iota(jnp.int32, (B, NB), 0) == ngr)
    nohc = ngc_ref[...] == jax.lax.broadcasted_iota(
        jnp.int32, (NB, B), 1)
    xcat = jnp.concatenate([xb, nohc.astype(bf16)], axis=1)
    Wncat = jnp.concatenate([WnT_ref[...], unb], axis=0)
    npre = jnp.dot(xcat, Wncat, preferred_element_type=f32)
    n1 = jnp.maximum(npre, 0.0).astype(bf16)
    nsum[...] += jnp.dot(nohr.astype(bf16), n1, preferred_element_type=f32)
    ncnt[...] += jnp.sum(nohr.astype(f32), axis=1, keepdims=True)

    @pl.when(i == G - 1)
    def _():
        navg = nsum[...] / jnp.maximum(ncnt[...], 1.0)
        eavg = esumT[...] / jnp.maximum(ecnt[...], 1.0)
        sv = (jnp.dot(navg, WgnT_ref[...], preferred_element_type=f32)
              + jnp.dot(eavg, WgeT_ref[...], preferred_element_type=f32)
              + bg_ref[...])
        sa = jnp.concatenate([sv, a_ref[...]], axis=1)
        h = jnp.maximum(jnp.dot(sa, WhT_ref[...], preferred_element_type=f32)
                        + bh_ref[...], 0.0)
        out_ref[...] = jnp.maximum(
            jnp.dot(h, WoT_ref[...], preferred_element_type=f32) + bo_ref[...], 0.0)


def kernel(x, edge_attr, u, a, node_gid, edge_gid, We, Weg, be, Wn, Wng, bn,
           Wgn, Wge, bg, Wh, bh, Wo, bo, *, interpret=False):
    N, dN = x.shape
    E, dE = edge_attr.shape
    B, dG = u.shape
    nA = a.shape[1]
    H = Wh.shape[0]
    G = 5
    assert E % G == 0 and N % G == 0
    EB = E // G
    NB = N // G

    f32 = jnp.float32
    bf16 = jnp.bfloat16
    eaT = edge_attr.T.astype(bf16)
    egid_row = edge_gid.reshape(G, 1, EB)
    ngid_row = node_gid.reshape(G, 1, NB)

    body = functools.partial(_body, G=G, B=B, EB=EB, NB=NB)
    const = lambda i: (0, 0)
    out = pl.pallas_call(
        body,
        grid=(G,),
        in_specs=[
            pl.BlockSpec((dE, EB), lambda i: (0, i)),
            pl.BlockSpec((1, 1, EB), lambda i: (i, 0, 0)),
            pl.BlockSpec((NB, dN), lambda i: (i, 0)),
            pl.BlockSpec((1, 1, NB), lambda i: (i, 0, 0)),
            pl.BlockSpec((NB, 1), lambda i: (i, 0)),
            pl.BlockSpec((dG, B), const),
            pl.BlockSpec((B, dG), const),
            pl.BlockSpec((128, dE), const),
            pl.BlockSpec((128, dG), const),
            pl.BlockSpec((128, 1), const),
            pl.BlockSpec((dN, 128), const),
            pl.BlockSpec((dG, 128), const),
            pl.BlockSpec((1, 128), const),
            pl.BlockSpec((128, 1), const),
            pl.BlockSpec((128, 1), const),
            pl.BlockSpec((1, 1), const),
            pl.BlockSpec((B, nA), const),
            pl.BlockSpec((1 + nA, H), const),
            pl.BlockSpec((1, H), const),
            pl.BlockSpec((H, 1), const),
            pl.BlockSpec((1, 1), const),
        ],
        out_specs=pl.BlockSpec((B, 1), const),
        out_shape=jax.ShapeDtypeStruct((B, 1), f32),
        scratch_shapes=[
            pltpu.VMEM((B, 128), f32),
            pltpu.VMEM((B, 1), f32),
            pltpu.VMEM((B, 128), f32),
            pltpu.VMEM((B, 1), f32),
        ],
        compiler_params=pltpu.CompilerParams(
            dimension_semantics=("arbitrary",)),
        interpret=interpret,
    )(eaT, egid_row, x.astype(bf16), ngid_row, node_gid.reshape(N, 1), u.T, u,
      We.astype(bf16), Weg, be.reshape(128, 1),
      Wn.T.astype(bf16), Wng.T, bn.reshape(1, 128),
      Wgn.T, Wge.T, bg.reshape(1, 1), a, Wh.T, bh.reshape(1, H),
      Wo.T, bo.reshape(1, 1))
    return out


# speedup vs baseline: 1.5327x; 1.0292x over previous
import functools

import jax
import jax.numpy as jnp
from jax.experimental import pallas as pl
from jax.experimental.pallas import tpu as pltpu


def _body(eaT_ref, egr_ref, x_ref, ngr_ref, ngc_ref, uT_ref, u_ref,
          We_ref, Weg_ref, beC_ref, WnT_ref, WngT_ref, bn_ref,
          WgnT_ref, WgeT_ref, bg_ref, a_ref, WhT_ref, bh_ref,
          WoT_ref, bo_ref, out_ref,
          esumT, ecnt, nsum, ncnt, *, G, B, EB, NB):
    i = pl.program_id(0)

    @pl.when(i == 0)
    def _():
        esumT[...] = jnp.zeros_like(esumT)
        ecnt[...] = jnp.zeros_like(ecnt)
        nsum[...] = jnp.zeros_like(nsum)
        ncnt[...] = jnp.zeros_like(ncnt)

    f32 = jnp.float32
    bf16 = jnp.bfloat16

    ugbT = (jnp.dot(Weg_ref[...], uT_ref[...], preferred_element_type=f32)
            + beC_ref[...]).astype(bf16)
    unb = (jnp.dot(u_ref[...], WngT_ref[...], preferred_element_type=f32)
           + bn_ref[...]).astype(bf16)

    ebT = eaT_ref[...]
    egr = egr_ref[0]
    ohr = (jax.lax.broadcasted_iota(jnp.int32, (B, EB), 0) == egr)
    ohr_b = ohr.astype(bf16)
    inT = jnp.concatenate([ebT, ohr_b], axis=0)
    Wfull = jnp.concatenate([We_ref[...], ugbT], axis=1)
    preT = jnp.dot(Wfull, inT, preferred_element_type=f32)
    e1T = jnp.maximum(preT, 0.0).astype(bf16)
    esumT[...] += jax.lax.dot_general(
        ohr_b, e1T, (((1,), (1,)), ((), ())), preferred_element_type=f32)
    ecnt[...] += jnp.sum(ohr.astype(f32), axis=1, keepdims=True)

    xb = x_ref[...].astype(bf16)
    ngr = ngr_ref[0]
    nohr = (jax.lax.broadcasted_iota(jnp.int32, (B, NB), 0) == ngr)
    nohc = ngc_ref[...] == jax.lax.broadcasted_iota(
        jnp.int32, (NB, B), 1)
    xcat = jnp.concatenate([xb, nohc.astype(bf16)], axis=1)
    Wncat = jnp.concatenate([WnT_ref[...], unb], axis=0)
    npre = jnp.dot(xcat, Wncat, preferred_element_type=f32)
    n1 = jnp.maximum(npre, 0.0).astype(bf16)
    nsum[...] += jnp.dot(nohr.astype(bf16), n1, preferred_element_type=f32)
    ncnt[...] += jnp.sum(nohr.astype(f32), axis=1, keepdims=True)

    @pl.when(i == G - 1)
    def _():
        navg = nsum[...] / jnp.maximum(ncnt[...], 1.0)
        eavg = esumT[...] / jnp.maximum(ecnt[...], 1.0)
        sv = (jnp.dot(navg, WgnT_ref[...], preferred_element_type=f32)
              + jnp.dot(eavg, WgeT_ref[...], preferred_element_type=f32)
              + bg_ref[...])
        sa = jnp.concatenate([sv, a_ref[...]], axis=1)
        h = jnp.maximum(jnp.dot(sa, WhT_ref[...], preferred_element_type=f32)
                        + bh_ref[...], 0.0)
        out_ref[...] = jnp.maximum(
            jnp.dot(h, WoT_ref[...], preferred_element_type=f32) + bo_ref[...], 0.0)


def kernel(x, edge_attr, u, a, node_gid, edge_gid, We, Weg, be, Wn, Wng, bn,
           Wgn, Wge, bg, Wh, bh, Wo, bo, *, interpret=False):
    N, dN = x.shape
    E, dE = edge_attr.shape
    B, dG = u.shape
    nA = a.shape[1]
    H = Wh.shape[0]
    G = 5
    assert E % G == 0 and N % G == 0
    EB = E // G
    NB = N // G

    f32 = jnp.float32
    bf16 = jnp.bfloat16
    eaT = edge_attr.T.astype(bf16)
    egid_row = edge_gid.reshape(G, 1, EB)
    ngid_row = node_gid.reshape(G, 1, NB)

    body = functools.partial(_body, G=G, B=B, EB=EB, NB=NB)
    const = lambda i: (0, 0)
    out = pl.pallas_call(
        body,
        grid=(G,),
        in_specs=[
            pl.BlockSpec((dE, EB), lambda i: (0, i)),
            pl.BlockSpec((1, 1, EB), lambda i: (i, 0, 0)),
            pl.BlockSpec((NB, dN), lambda i: (i, 0)),
            pl.BlockSpec((1, 1, NB), lambda i: (i, 0, 0)),
            pl.BlockSpec((NB, 1), lambda i: (i, 0)),
            pl.BlockSpec((dG, B), const),
            pl.BlockSpec((B, dG), const),
            pl.BlockSpec((128, dE), const),
            pl.BlockSpec((128, dG), const),
            pl.BlockSpec((128, 1), const),
            pl.BlockSpec((dN, 128), const),
            pl.BlockSpec((dG, 128), const),
            pl.BlockSpec((1, 128), const),
            pl.BlockSpec((128, 1), const),
            pl.BlockSpec((128, 1), const),
            pl.BlockSpec((1, 1), const),
            pl.BlockSpec((B, nA), const),
            pl.BlockSpec((1 + nA, H), const),
            pl.BlockSpec((1, H), const),
            pl.BlockSpec((H, 1), const),
            pl.BlockSpec((1, 1), const),
        ],
        out_specs=pl.BlockSpec((B, 1), const),
        out_shape=jax.ShapeDtypeStruct((B, 1), f32),
        scratch_shapes=[
            pltpu.VMEM((B, 128), f32),
            pltpu.VMEM((B, 1), f32),
            pltpu.VMEM((B, 128), f32),
            pltpu.VMEM((B, 1), f32),
        ],
        compiler_params=pltpu.CompilerParams(
            dimension_semantics=("arbitrary",)),
        interpret=interpret,
    )(eaT, egid_row, x, ngid_row, node_gid.reshape(N, 1), u.T, u,
      We.astype(bf16), Weg, be.reshape(128, 1),
      Wn.T.astype(bf16), Wng.T, bn.reshape(1, 128),
      Wgn.T, Wge.T, bg.reshape(1, 1), a, Wh.T, bh.reshape(1, H),
      Wo.T, bo.reshape(1, 1))
    return out
